# baseline (speedup 1.0000x reference)
import jax
import jax.numpy as jnp
from jax.experimental import pallas as pl
from jax.experimental.pallas import tpu as pltpu

_BLK = 400_000


def _body(x_ref, out_ref):
    i = pl.program_id(0)
    j = pl.program_id(1)

    @pl.when(jnp.logical_and(i == 0, j == 0))
    def _init():
        out_ref[0, 0] = 0

    nz = jnp.any(x_ref[...] != 0.0).astype(jnp.int32)
    out_ref[0, 0] = out_ref[0, 0] | nz


def kernel(x):
    r, c, n = x.shape
    xf = x.reshape(r * c, 1, n)
    res = pl.pallas_call(
        _body,
        grid=(2, n // _BLK),
        in_specs=[pl.BlockSpec((1, 1, _BLK), lambda i, j: (c * (i + 1), 0, j))],
        out_specs=pl.BlockSpec(memory_space=pltpu.SMEM),
        out_shape=jax.ShapeDtypeStruct((1, 1), jnp.int32),
    )(xf)
    return (res[0, 0] != 0).reshape(1)


# speedup vs baseline: 8.8642x; 8.8642x over previous
import jax
import jax.numpy as jnp
from jax.experimental import pallas as pl
from jax.experimental.pallas import tpu as pltpu

_CH = 400_000


def _body(x_hbm, out_ref, scr, sem):
    step = pl.program_id(0)
    r = 1 + step // 5
    j = step % 5

    @pl.when(step == 0)
    def _init():
        out_ref[0, 0] = 0

    cp = pltpu.make_async_copy(x_hbm.at[r, 0, pl.ds(j * _CH, _CH)], scr, sem)
    cp.start()
    cp.wait()
    nz = jnp.any(scr[...] != 0.0).astype(jnp.int32)
    out_ref[0, 0] = out_ref[0, 0] | nz


def kernel(x):
    res = pl.pallas_call(
        _body,
        grid=(10,),
        in_specs=[pl.BlockSpec(memory_space=pl.ANY)],
        out_specs=pl.BlockSpec(memory_space=pltpu.SMEM),
        out_shape=jax.ShapeDtypeStruct((1, 1), jnp.int32),
        scratch_shapes=[
            pltpu.VMEM((_CH,), jnp.float32),
            pltpu.SemaphoreType.DMA,
        ],
    )(x)
    return (res[0, 0] != 0).reshape(1)


# speedup vs baseline: 47.2774x; 5.3335x over previous
import jax
import jax.numpy as jnp
from jax.experimental import pallas as pl
from jax.experimental.pallas import tpu as pltpu

_BLK = 80_000


def _body(x_ref, out_ref):
    t = pl.program_id(0)

    @pl.when(t == 0)
    def _init():
        out_ref[0, 0] = 0

    nz = jnp.any(x_ref[0, 1:3, :] != 0.0).astype(jnp.int32)
    out_ref[0, 0] = out_ref[0, 0] | nz


def kernel(x):
    n = x.shape[2]
    xt = jnp.swapaxes(x, 0, 1)
    res = pl.pallas_call(
        _body,
        grid=(n // _BLK,),
        in_specs=[pl.BlockSpec((1, 3, _BLK), lambda t: (0, 0, t))],
        out_specs=pl.BlockSpec(memory_space=pltpu.SMEM),
        out_shape=jax.ShapeDtypeStruct((1, 1), jnp.int32),
    )(xt)
    return (res[0, 0] != 0).reshape(1)


# speedup vs baseline: 53.8422x; 1.1389x over previous
import jax
import jax.numpy as jnp
from jax.experimental import pallas as pl
from jax.experimental.pallas import tpu as pltpu

_CH = 80_000
_NCH = 25


def _body(x_hbm, out_ref, buf, sems):
    t = pl.program_id(0)

    def _cp(idx):
        return pltpu.make_async_copy(
            x_hbm.at[0, pl.ds(1, 2), pl.ds(idx * _CH, _CH)],
            buf.at[idx % 2],
            sems.at[idx % 2],
        )

    @pl.when(t == 0)
    def _init():
        out_ref[0, 0] = 0
        _cp(0).start()

    @pl.when(t + 1 < _NCH)
    def _prefetch():
        _cp(t + 1).start()

    _cp(t).wait()
    nz = jnp.any(buf[t % 2] != 0.0).astype(jnp.int32)
    out_ref[0, 0] = out_ref[0, 0] | nz


def kernel(x):
    xt = jnp.swapaxes(x, 0, 1)
    res = pl.pallas_call(
        _body,
        grid=(_NCH,),
        in_specs=[pl.BlockSpec(memory_space=pl.ANY)],
        out_specs=pl.BlockSpec(memory_space=pltpu.SMEM),
        out_shape=jax.ShapeDtypeStruct((1, 1), jnp.int32),
        scratch_shapes=[
            pltpu.VMEM((2, 2, _CH), jnp.float32),
            pltpu.SemaphoreType.DMA((2,)),
        ],
    )(xt)
    return (res[0, 0] != 0).reshape(1)
